# baseline (speedup 1.0000x reference)
import functools

import jax
import jax.numpy as jnp
from jax.experimental import pallas as pl

NUM_HEADS = 8
NUM_EMBEDDINGS = 512
EMBEDDING_DIM = 256
HEAD_DIM = EMBEDDING_DIM // NUM_HEADS
GROUPS = 2
HEADS_PER_GROUP = NUM_HEADS // GROUPS
GROUP_DIM = HEADS_PER_GROUP * HEAD_DIM
GROUP_EMB = HEADS_PER_GROUP * NUM_EMBEDDINGS


def _pq_kernel(z_ref, cbtg_ref, mask_ref, csq_ref, cb_ref,
               zq_ref, idxp_ref, dist_ref):
    p = pl.program_id(1)
    zg = z_ref[:, pl.ds(p * GROUP_DIM, GROUP_DIM)]
    dotg = jnp.dot(zg, cbtg_ref[p], preferred_element_type=jnp.float32)
    zsqb = jnp.dot(zg * zg, mask_ref[...],
                   preferred_element_type=jnp.float32)
    dist = (zsqb + csq_ref[p][None, :]) + dotg
    dist_ref[...] = dist
    idx_cols = []
    zq_parts = []
    for j in range(HEADS_PER_GROUP):
        cols = slice(j * NUM_EMBEDDINGS, (j + 1) * NUM_EMBEDDINGS)
        m = jnp.min(dist_ref[:, cols], axis=-1, keepdims=True)
        d = dist_ref[:, cols]
        iota = jax.lax.broadcasted_iota(jnp.int32, d.shape, 1)
        idx = jnp.min(jnp.where(d == m, iota, NUM_EMBEDDINGS), axis=-1)
        idx_cols.append(idx[:, None].astype(jnp.int32))
        onehot = (iota == idx[:, None]).astype(jnp.float32)
        zq_j = jnp.dot(onehot, cb_ref[p * HEADS_PER_GROUP + j],
                       preferred_element_type=jnp.float32)
        zh = zg[:, j * HEAD_DIM:(j + 1) * HEAD_DIM]
        zq_parts.append(zh + (zq_j - zh))
    idxp_ref[0] = jnp.concatenate(idx_cols, axis=1)
    zq_ref[:, pl.ds(p * GROUP_DIM, GROUP_DIM)] = jnp.concatenate(zq_parts,
                                                                 axis=1)


@functools.partial(jax.jit, static_argnames=("block_b",))
def _pq(z, codebooks, block_b=1024):
    bsz, dim = z.shape
    cbt = jnp.transpose(codebooks, (0, 2, 1))
    cbtg = jnp.zeros((GROUPS, HEADS_PER_GROUP, HEAD_DIM,
                      HEADS_PER_GROUP, NUM_EMBEDDINGS), jnp.float32)
    cbtr = cbt.reshape(GROUPS, HEADS_PER_GROUP, HEAD_DIM, NUM_EMBEDDINGS)
    for j in range(HEADS_PER_GROUP):
        cbtg = cbtg.at[:, j, :, j, :].set(-2.0 * cbtr[:, j])
    cbtg = cbtg.reshape(GROUPS, GROUP_DIM, GROUP_EMB)
    mask = (jax.lax.broadcasted_iota(jnp.int32, (GROUP_DIM, GROUP_EMB), 0)
            // HEAD_DIM ==
            jax.lax.broadcasted_iota(jnp.int32, (GROUP_DIM, GROUP_EMB), 1)
            // NUM_EMBEDDINGS).astype(jnp.float32)
    csq = jnp.sum(codebooks ** 2, axis=-1).reshape(GROUPS, GROUP_EMB)
    grid = (bsz // block_b, GROUPS)
    zq, idxp, dist = pl.pallas_call(
        _pq_kernel,
        grid=grid,
        in_specs=[
            pl.BlockSpec((block_b, dim), lambda i, p: (i, 0)),
            pl.BlockSpec((GROUPS, GROUP_DIM, GROUP_EMB),
                         lambda i, p: (0, 0, 0)),
            pl.BlockSpec((GROUP_DIM, GROUP_EMB), lambda i, p: (0, 0)),
            pl.BlockSpec((GROUPS, GROUP_EMB), lambda i, p: (0, 0)),
            pl.BlockSpec((NUM_HEADS, NUM_EMBEDDINGS, HEAD_DIM),
                         lambda i, p: (0, 0, 0)),
        ],
        out_specs=[
            pl.BlockSpec((block_b, dim), lambda i, p: (i, 0)),
            pl.BlockSpec((1, block_b, HEADS_PER_GROUP), lambda i, p: (p, i, 0)),
            pl.BlockSpec((block_b, GROUP_EMB), lambda i, p: (i, p)),
        ],
        out_shape=[
            jax.ShapeDtypeStruct((bsz, dim), jnp.float32),
            jax.ShapeDtypeStruct((GROUPS, bsz, HEADS_PER_GROUP), jnp.int32),
            jax.ShapeDtypeStruct((bsz, NUM_HEADS * NUM_EMBEDDINGS), jnp.float32),
        ],
    )(z, cbtg, mask, csq, codebooks)
    idx = jnp.transpose(idxp, (1, 0, 2)).reshape(bsz, NUM_HEADS)
    return zq, idx, dist.reshape(bsz, NUM_HEADS, NUM_EMBEDDINGS)


def kernel(z, codebooks):
    return _pq(z, codebooks)
